# baseline (speedup 1.0000x reference)
import functools

import jax
import jax.numpy as jnp
from jax import lax
from jax.experimental import pallas as pl
from jax.experimental.pallas import tpu as pltpu
from jax.experimental.pallas import tpu_sc as plsc

_N = 100000
_D = 128
_NG = 64
_L = 16
_NC = 2
_NS = 16
_P = 100352
_PADLEN = _P + _L
_C1 = _P // _NS
_C2 = _P // (_NC * _NS)
_T = 80


def _scale_body(batch_hbm, scale_hbm, bchunk, lcum, idxv, ttable, pbuf,
                bchunk2, outchunk, shared):
    c = lax.axis_index("c")
    s = lax.axis_index("s")
    zeros16 = jnp.zeros((_L,), jnp.int32)
    iota16 = lax.iota(jnp.int32, _L)

    for k in range(_T // _L):
        lcum[pl.ds(k * _L, _L)] = zeros16
        idxv[pl.ds(k * _L, _L)] = iota16 + (k * _L)

    @pl.when(s == 0)
    def _zero_shared():
        pltpu.sync_copy(lcum, shared)

    start1 = s * _C1
    pltpu.sync_copy(batch_hbm.at[pl.ds(start1, _C1 + _L)], bchunk)

    def _scan_step(k, carry):
        bv = bchunk[pl.ds(k * _L, _L)]
        bnext = plsc.load_gather(bchunk, [iota16 + (k * _L + 1)])
        mask = bv != bnext
        endpos = iota16 + (start1 + k * _L + 1)
        plsc.store_scatter(lcum, [bv], endpos, mask=mask)
        return carry

    lax.fori_loop(0, _C1 // _L, _scan_step, 0)

    plsc.subcore_barrier()
    pltpu.sync_copy(lcum, shared.at[idxv], add=True)
    plsc.subcore_barrier()
    pltpu.sync_copy(shared, lcum)

    pbuf[pl.ds(0, _L)] = zeros16
    carry = jnp.int32(0)
    for k in range(_NG // _L):
        cv = lcum[pl.ds(k * _L, _L)]
        cm = jnp.maximum(plsc.cummax(cv), carry)
        carry = jnp.max(cm)
        plsc.store_scatter(pbuf, [iota16 + (k * _L + 1)], cm)
        prev = pbuf[pl.ds(k * _L, _L)]
        deg = (cm - prev).astype(jnp.float32)
        yi = jnp.int32(0x5F3759DF) - (plsc.bitcast(deg, jnp.int32) >> 1)
        y = plsc.bitcast(yi, jnp.float32)
        for _ in range(3):
            y = y * (1.5 - 0.5 * deg * y * y)
        ttable[pl.ds(k * _L, _L)] = y
    for k in range(_NG // _L, _T // _L):
        ttable[pl.ds(k * _L, _L)] = jnp.zeros((_L,), jnp.float32)

    w = c * _NS + s
    start2 = w * _C2
    pltpu.sync_copy(batch_hbm.at[pl.ds(start2, _C2)], bchunk2)

    def _gather_step(k, carry):
        bv = bchunk2[pl.ds(k * _L, _L)]
        outchunk[pl.ds(k * _L, _L)] = plsc.load_gather(ttable, [bv])
        return carry

    lax.fori_loop(0, _C2 // _L, _gather_step, 0)
    pltpu.sync_copy(outchunk, scale_hbm.at[pl.ds(start2, _C2)])


def _node_scale(batch_pad):
    mesh = plsc.VectorSubcoreMesh(core_axis_name="c", subcore_axis_name="s")
    f = functools.partial(
        pl.kernel,
        mesh=mesh,
        out_type=jax.ShapeDtypeStruct((_P,), jnp.float32),
        compiler_params=pltpu.CompilerParams(needs_layout_passes=False),
        scratch_types=[
            pltpu.VMEM((_C1 + _L,), jnp.int32),
            pltpu.VMEM((_T,), jnp.int32),
            pltpu.VMEM((_T,), jnp.int32),
            pltpu.VMEM((_T,), jnp.float32),
            pltpu.VMEM((_T,), jnp.int32),
            pltpu.VMEM((_C2,), jnp.int32),
            pltpu.VMEM((_C2,), jnp.float32),
            pltpu.VMEM_SHARED((_T,), jnp.int32),
        ],
    )(_scale_body)
    return f(batch_pad)


def _mul_body(x_ref, s_ref, o_ref):
    o_ref[...] = x_ref[...] * s_ref[...]


_BLK = 2000


def _scaled_mul(x, scale2d):
    return pl.pallas_call(
        _mul_body,
        grid=(_N // _BLK,),
        in_specs=[
            pl.BlockSpec((_BLK, _D), lambda i: (i, 0)),
            pl.BlockSpec((_BLK, 1), lambda i: (i, 0)),
        ],
        out_specs=pl.BlockSpec((_BLK, _D), lambda i: (i, 0)),
        out_shape=jax.ShapeDtypeStruct((_N, _D), jnp.float32),
    )(x, scale2d)


def kernel(x, batch):
    b32 = batch.astype(jnp.int32)
    batch_pad = jnp.concatenate(
        [b32, jnp.full((_PADLEN - _N,), _NG, jnp.int32)])
    scale = _node_scale(batch_pad)
    return _scaled_mul(x, scale[:_N].reshape(_N, 1))


# speedup vs baseline: 1.1393x; 1.1393x over previous
import functools

import jax
import jax.numpy as jnp
from jax import lax
from jax.experimental import pallas as pl
from jax.experimental.pallas import tpu as pltpu
from jax.experimental.pallas import tpu_sc as plsc

_N = 100000
_D = 128
_NG = 64
_L = 16
_NC = 2
_NS = 16
_P = 100352
_PADLEN = _P + _L
_C1 = _P // _NS
_C2 = _P // (_NC * _NS)
_T = 80


def _scale_body(batch_hbm, scale_hbm, bchunk, lcum, idxv, ttable, pbuf,
                bchunk2, outchunk, shared):
    c = lax.axis_index("c")
    s = lax.axis_index("s")
    zeros16 = jnp.zeros((_L,), jnp.int32)
    iota16 = lax.iota(jnp.int32, _L)

    for k in range(_T // _L):
        lcum[pl.ds(k * _L, _L)] = zeros16
        idxv[pl.ds(k * _L, _L)] = iota16 + (k * _L)

    @pl.when(s == 0)
    def _zero_shared():
        pltpu.sync_copy(lcum, shared)

    start1 = s * _C1
    pltpu.sync_copy(batch_hbm.at[pl.ds(start1, _C1 + _L)], bchunk)

    def _scan_step(k, carry):
        bv = bchunk[pl.ds(k * _L, _L)]
        bnext = plsc.load_gather(bchunk, [iota16 + (k * _L + 1)])
        mask = bv != bnext
        endpos = iota16 + (start1 + k * _L + 1)
        plsc.store_scatter(lcum, [bv], endpos, mask=mask)
        return carry

    lax.fori_loop(0, _C1 // _L, _scan_step, 0)

    plsc.subcore_barrier()
    pltpu.sync_copy(lcum, shared.at[idxv], add=True)
    plsc.subcore_barrier()
    pltpu.sync_copy(shared, lcum)

    pbuf[pl.ds(0, _L)] = zeros16
    carry = jnp.int32(0)
    for k in range(_NG // _L):
        cv = lcum[pl.ds(k * _L, _L)]
        cm = jnp.maximum(plsc.cummax(cv), carry)
        carry = jnp.max(cm)
        plsc.store_scatter(pbuf, [iota16 + (k * _L + 1)], cm)
        prev = pbuf[pl.ds(k * _L, _L)]
        deg = (cm - prev).astype(jnp.float32)
        yi = jnp.int32(0x5F3759DF) - (plsc.bitcast(deg, jnp.int32) >> 1)
        y = plsc.bitcast(yi, jnp.float32)
        for _ in range(3):
            y = y * (1.5 - 0.5 * deg * y * y)
        ttable[pl.ds(k * _L, _L)] = y
    for k in range(_NG // _L, _T // _L):
        ttable[pl.ds(k * _L, _L)] = jnp.zeros((_L,), jnp.float32)

    w = c * _NS + s
    start2 = w * _C2
    pltpu.sync_copy(batch_hbm.at[pl.ds(start2, _C2)], bchunk2)

    def _gather_step(k, carry):
        bv = bchunk2[pl.ds(k * _L, _L)]
        outchunk[pl.ds(k * _L, _L)] = plsc.load_gather(ttable, [bv])
        return carry

    lax.fori_loop(0, _C2 // _L, _gather_step, 0)
    pltpu.sync_copy(outchunk, scale_hbm.at[pl.ds(start2, _C2)])


def _node_scale(batch_pad):
    mesh = plsc.VectorSubcoreMesh(core_axis_name="c", subcore_axis_name="s")
    f = functools.partial(
        pl.kernel,
        mesh=mesh,
        out_type=jax.ShapeDtypeStruct((_P,), jnp.float32),
        compiler_params=pltpu.CompilerParams(needs_layout_passes=False),
        scratch_types=[
            pltpu.VMEM((_C1 + _L,), jnp.int32),
            pltpu.VMEM((_T,), jnp.int32),
            pltpu.VMEM((_T,), jnp.int32),
            pltpu.VMEM((_T,), jnp.float32),
            pltpu.VMEM((_T,), jnp.int32),
            pltpu.VMEM((_C2,), jnp.int32),
            pltpu.VMEM((_C2,), jnp.float32),
            pltpu.VMEM_SHARED((_T,), jnp.int32),
        ],
    )(_scale_body)
    return f(batch_pad)


def _mul_body(x_ref, s_ref, o_ref):
    o_ref[...] = x_ref[...] * s_ref[...]


_BLK = 10000


def _scaled_mul(x, scale2d):
    return pl.pallas_call(
        _mul_body,
        grid=(_N // _BLK,),
        in_specs=[
            pl.BlockSpec((_BLK, _D), lambda i: (i, 0)),
            pl.BlockSpec((_BLK, 1), lambda i: (i, 0)),
        ],
        out_specs=pl.BlockSpec((_BLK, _D), lambda i: (i, 0)),
        out_shape=jax.ShapeDtypeStruct((_N, _D), jnp.float32),
    )(x, scale2d)


def kernel(x, batch):
    b32 = batch.astype(jnp.int32)
    batch_pad = jnp.concatenate(
        [b32, jnp.full((_PADLEN - _N,), _NG, jnp.int32)])
    scale = _node_scale(batch_pad)
    return _scaled_mul(x, scale[:_N].reshape(_N, 1))
